# baseline (speedup 1.0000x reference)
import jax
import jax.numpy as jnp
from jax.experimental import pallas as pl
from jax.experimental.pallas import tpu as pltpu

R, C = 128, 32768


def _body(src_ref, scale_ref, x_ref, o_ref):
    i = pl.program_id(0)
    o_ref[...] = x_ref[...] * scale_ref[i]


def kernel(x, mask):
    ar = jnp.arange(R, dtype=jnp.int32)
    src = jnp.where(mask != 0.0, ar, ar ^ 1)
    grid_spec = pltpu.PrefetchScalarGridSpec(
        num_scalar_prefetch=2,
        grid=(R,),
        in_specs=[
            pl.BlockSpec((1, 1, C), lambda i, src, scale: (src[i], 0, 0)),
        ],
        out_specs=pl.BlockSpec((1, 1, C), lambda i, src, scale: (i, 0, 0)),
    )
    out = pl.pallas_call(
        _body,
        grid_spec=grid_spec,
        out_shape=jax.ShapeDtypeStruct((R, 1, C), x.dtype),
    )(src, mask, x.reshape(R, 1, C))
    return out.reshape(R, C)
